# Initial kernel scaffold; baseline (speedup 1.0000x reference)
import jax
import jax.numpy as jnp
from jax.experimental import pallas as pl


def kernel(x, table):
    raise NotImplementedError("write your pallas kernel here")


# speedup vs baseline: 1.2757x; 1.2757x over previous
import functools

import jax
import jax.numpy as jnp
from jax import lax
from jax.experimental import pallas as pl
from jax.experimental.pallas import tpu as pltpu
from jax.experimental.pallas import tpu_sc as plsc

_B = 16384
_F = 26
_FIELD_DIM = 40000
_NC = 2
_NS = 16
_NW = _NC * _NS
_RPW = _B // _NW
_EPW = _RPW * _F
_L = 16
_BBLK = _RPW // _L


def _tec_body(x_hbm, table_hbm, out_hbm, xv, tv, valv, outv, sem):
    wid = lax.axis_index("s") * _NC + lax.axis_index("c")
    base_elem = wid * _EPW
    base_row = wid * _RPW

    pltpu.sync_copy(x_hbm.at[pl.ds(base_elem, _EPW)], xv)

    iota26 = lax.iota(jnp.int32, _L) * _F

    def build(bb, _):
        src_base = iota26 + bb * (_L * _F)
        for f in range(_F):
            vals = plsc.load_gather(xv, [src_base + f])
            tv[pl.ds(f * _RPW + bb * _L, _L)] = vals + f * _FIELD_DIM
        return 0

    lax.fori_loop(0, _BBLK, build, 0, unroll=False)

    pltpu.async_copy(table_hbm.at[tv], valv, sem).wait()

    def reduce(bb, _):
        acc = valv[pl.ds(bb * _L, _L)]
        for f in range(1, _F):
            acc = acc + valv[pl.ds(f * _RPW + bb * _L, _L)]
        outv[pl.ds(bb * _L, _L)] = acc
        return 0

    lax.fori_loop(0, _BBLK, reduce, 0, unroll=False)

    pltpu.sync_copy(outv, out_hbm.at[pl.ds(base_row, _RPW)])


@jax.jit
def _fmlinear(x_flat, table_flat):
    mesh = plsc.VectorSubcoreMesh(
        core_axis_name="c", subcore_axis_name="s",
        num_cores=_NC, num_subcores=_NS,
    )
    return pl.kernel(
        _tec_body,
        out_type=jax.ShapeDtypeStruct((_B,), jnp.float32),
        mesh=mesh,
        scratch_types=[
            pltpu.VMEM((_EPW,), jnp.int32),
            pltpu.VMEM((_EPW,), jnp.int32),
            pltpu.VMEM((_EPW,), jnp.float32),
            pltpu.VMEM((_RPW,), jnp.float32),
            pltpu.SemaphoreType.DMA,
        ],
        compiler_params=pltpu.CompilerParams(needs_layout_passes=False),
    )(x_flat, table_flat)


def kernel(x, table):
    out = _fmlinear(x.reshape(-1), table.reshape(-1))
    return out.reshape(_B, 1)


# speedup vs baseline: 2.5812x; 2.0234x over previous
import jax
import jax.numpy as jnp
from jax import lax
from jax.experimental import pallas as pl
from jax.experimental.pallas import tpu as pltpu
from jax.experimental.pallas import tpu_sc as plsc

_B = 16384
_F = 26
_FIELD_DIM = 40000
_NC = 2
_NS = 16
_NW = _NC * _NS
_RPW = _B // _NW
_EPW = _RPW * _F
_L = 16
_BBLK = _RPW // _L


def _tec_body(x_hbm, table_hbm, out_hbm, xv, valv, outv, sem):
    wid = lax.axis_index("s") * _NC + lax.axis_index("c")
    base_row = wid * _RPW

    copies = [
        pltpu.async_copy(
            x_hbm.at[pl.ds(f * _B + base_row, _RPW)],
            xv.at[pl.ds(f * _RPW, _RPW)],
            sem,
        )
        for f in range(_F)
    ]
    for cp in copies:
        cp.wait()

    def addoff(bb, _):
        base = bb * _L
        for f in range(1, _F):
            s = pl.ds(f * _RPW + base, _L)
            xv[s] = xv[s] + f * _FIELD_DIM
        return 0

    lax.fori_loop(0, _BBLK, addoff, 0)

    pltpu.async_copy(table_hbm.at[xv], valv, sem).wait()

    def reduce(bb, _):
        base = bb * _L
        acc = valv[pl.ds(base, _L)]
        for f in range(1, _F):
            acc = acc + valv[pl.ds(f * _RPW + base, _L)]
        outv[pl.ds(base, _L)] = acc
        return 0

    lax.fori_loop(0, _BBLK, reduce, 0)

    pltpu.sync_copy(outv, out_hbm.at[pl.ds(base_row, _RPW)])


_TPAD = 1040384


@jax.jit
def _fmlinear(x_t_flat, table_flat):
    mesh = plsc.VectorSubcoreMesh(
        core_axis_name="c", subcore_axis_name="s",
        num_cores=_NC, num_subcores=_NS,
    )
    return pl.kernel(
        _tec_body,
        out_type=jax.ShapeDtypeStruct((_B,), jnp.float32),
        mesh=mesh,
        scratch_types=[
            pltpu.VMEM((_EPW,), jnp.int32),
            pltpu.VMEM((_EPW,), jnp.float32),
            pltpu.VMEM((_RPW,), jnp.float32),
            pltpu.SemaphoreType.DMA,
        ],
        compiler_params=pltpu.CompilerParams(needs_layout_passes=False),
    )(x_t_flat, table_flat)


def kernel(x, table):
    table_flat = jnp.pad(table, ((0, _TPAD - table.shape[0]), (0, 0))).reshape(-1)
    out = _fmlinear(x.T.reshape(-1), table_flat)
    return out.reshape(_B, 1)


# speedup vs baseline: 2.7866x; 1.0796x over previous
import jax
import jax.numpy as jnp
from jax import lax
from jax.experimental import pallas as pl
from jax.experimental.pallas import tpu as pltpu
from jax.experimental.pallas import tpu_sc as plsc

_B = 16384
_F = 26
_FIELD_DIM = 40000
_NC = 2
_NS = 16
_NW = _NC * _NS
_RPW = _B // _NW
_EPW = _RPW * _F
_L = 16
_BBLK = _RPW // _L


def _tec_body(x_hbm, table_hbm, out_hbm, xv, valv, outv, sem):
    wid = lax.axis_index("s") * _NC + lax.axis_index("c")
    base_row = wid * _RPW

    copies = [
        pltpu.async_copy(
            x_hbm.at[f, pl.ds(base_row, _RPW)],
            xv.at[pl.ds(f * _RPW, _RPW)],
            sem,
        )
        for f in range(_F)
    ]
    for cp in copies:
        cp.wait()

    def addoff(bb, _):
        base = bb * _L
        for f in range(1, _F):
            s = pl.ds(f * _RPW + base, _L)
            xv[s] = xv[s] + f * _FIELD_DIM
        return 0

    lax.fori_loop(0, _BBLK, addoff, 0)

    pltpu.async_copy(table_hbm.at[xv], valv, sem).wait()

    def reduce(bb, _):
        base = bb * _L
        acc = valv[pl.ds(base, _L)]
        for f in range(1, _F):
            acc = acc + valv[pl.ds(f * _RPW + base, _L)]
        outv[pl.ds(base, _L)] = acc
        return 0

    lax.fori_loop(0, _BBLK, reduce, 0)

    pltpu.sync_copy(outv, out_hbm.at[pl.ds(base_row, _RPW)])


_TPAD = 1040384


@jax.jit
def _fmlinear(x_t, table_flat):
    mesh = plsc.VectorSubcoreMesh(
        core_axis_name="c", subcore_axis_name="s",
        num_cores=_NC, num_subcores=_NS,
    )
    return pl.kernel(
        _tec_body,
        out_type=jax.ShapeDtypeStruct((_B,), jnp.float32),
        mesh=mesh,
        scratch_types=[
            pltpu.VMEM((_EPW,), jnp.int32),
            pltpu.VMEM((_EPW,), jnp.float32),
            pltpu.VMEM((_RPW,), jnp.float32),
            pltpu.SemaphoreType.DMA,
        ],
        compiler_params=pltpu.CompilerParams(
            needs_layout_passes=False, disable_bounds_checks=True,
        ),
    )(x_t, table_flat)


def kernel(x, table):
    table_flat = jnp.pad(table, ((0, _TPAD - table.shape[0]), (0, 0))).reshape(-1)
    out = _fmlinear(x.T, table_flat)
    return out.reshape(_B, 1)


# speedup vs baseline: 3.2772x; 1.1761x over previous
import jax
import jax.numpy as jnp
from jax import lax
from jax.experimental import pallas as pl
from jax.experimental.pallas import tpu as pltpu
from jax.experimental.pallas import tpu_sc as plsc

_B = 16384
_F = 26
_FIELD_DIM = 40000
_NC = 2
_NS = 16
_NW = _NC * _NS
_RPW = _B // _NW
_EPW = _RPW * _F
_L = 16
_BBLK = _RPW // _L


def _tec_body(x_hbm, table_hbm, out_hbm, xv, valv, outv, table_sp, sem, tsem):
    sid = lax.axis_index("s")
    wid = sid * _NC + lax.axis_index("c")
    base_row = wid * _RPW

    @pl.when(sid == 0)
    def _stage_table():
        pltpu.async_copy(table_hbm, table_sp, tsem).wait()

    copies = [
        pltpu.async_copy(
            x_hbm.at[f, pl.ds(base_row, _RPW)],
            xv.at[pl.ds(f * _RPW, _RPW)],
            sem,
        )
        for f in range(_F)
    ]
    for cp in copies:
        cp.wait()

    def addoff(bb, _):
        base = bb * _L
        for f in range(1, _F):
            s = pl.ds(f * _RPW + base, _L)
            xv[s] = xv[s] + f * _FIELD_DIM
        return 0

    lax.fori_loop(0, _BBLK, addoff, 0)

    plsc.subcore_barrier()
    pltpu.async_copy(table_sp.at[xv], valv, sem).wait()

    def reduce(bb, _):
        base = bb * _L
        acc = valv[pl.ds(base, _L)]
        for f in range(1, _F):
            acc = acc + valv[pl.ds(f * _RPW + base, _L)]
        outv[pl.ds(base, _L)] = acc
        return 0

    lax.fori_loop(0, _BBLK, reduce, 0)

    pltpu.sync_copy(outv, out_hbm.at[pl.ds(base_row, _RPW)])


_TPAD = 1040384


@jax.jit
def _fmlinear(x_t, table_flat):
    mesh = plsc.VectorSubcoreMesh(
        core_axis_name="c", subcore_axis_name="s",
        num_cores=_NC, num_subcores=_NS,
    )
    return pl.kernel(
        _tec_body,
        out_type=jax.ShapeDtypeStruct((_B,), jnp.float32),
        mesh=mesh,
        scratch_types=[
            pltpu.VMEM((_EPW,), jnp.int32),
            pltpu.VMEM((_EPW,), jnp.float32),
            pltpu.VMEM((_RPW,), jnp.float32),
            pltpu.VMEM_SHARED((_TPAD,), jnp.float32),
            pltpu.SemaphoreType.DMA,
            pltpu.SemaphoreType.DMA,
        ],
        compiler_params=pltpu.CompilerParams(
            needs_layout_passes=False, disable_bounds_checks=True,
        ),
    )(x_t, table_flat)


def kernel(x, table):
    table_flat = jnp.pad(table, ((0, _TPAD - table.shape[0]), (0, 0))).reshape(-1)
    out = _fmlinear(x.T, table_flat)
    return out.reshape(_B, 1)


# speedup vs baseline: 3.4643x; 1.0571x over previous
import jax
import jax.numpy as jnp
from jax import lax
from jax.experimental import pallas as pl
from jax.experimental.pallas import tpu as pltpu
from jax.experimental.pallas import tpu_sc as plsc

_B = 16384
_F = 26
_FIELD_DIM = 40000
_NC = 2
_NS = 16
_NW = _NC * _NS
_RPW = _B // _NW
_EPW = _RPW * _F
_L = 16
_BBLK = _RPW // _L
_FSP = 16
_CHW = 40960
_SPW = _FSP * _CHW
_TPAD = 1040384


def _tec_body(x_hbm, table_hbm, out_hbm, xv, valv, outv, table_sp, sem, gsem, hsem, tsem):
    sid = lax.axis_index("s")
    wid = sid * _NC + lax.axis_index("c")
    base_row = wid * _RPW

    for f in range(_FSP):
        @pl.when(sid == f)
        def _stage(f=f):
            pltpu.async_copy(
                table_hbm.at[pl.ds(f * _CHW, _CHW)],
                table_sp.at[pl.ds(f * _CHW, _CHW)],
                tsem,
            )

    xcopies = [
        pltpu.async_copy(
            x_hbm.at[f, pl.ds(base_row, _RPW)],
            xv.at[pl.ds(f * _RPW, _RPW)],
            sem,
        )
        for f in range(_F)
    ]

    for cp in xcopies:
        cp.wait()
    hcopies = []
    for f in range(_FSP, _F):
        hcopies.append(pltpu.async_copy(
            table_hbm.at[pl.ds(f * _FIELD_DIM, _FIELD_DIM)].at[
                xv.at[pl.ds(f * _RPW, _RPW)]],
            valv.at[pl.ds(f * _RPW, _RPW)],
            hsem,
        ))

    @pl.when(sid < _FSP)
    def _drain_stage():
        pltpu.make_async_copy(
            table_hbm.at[pl.ds(0, _CHW)],
            table_sp.at[pl.ds(0, _CHW)],
            tsem,
        ).wait()

    plsc.subcore_barrier()
    scopies = []
    for f in range(_FSP):
        scopies.append(pltpu.async_copy(
            table_sp.at[pl.ds(f * _FIELD_DIM, _FIELD_DIM)].at[
                xv.at[pl.ds(f * _RPW, _RPW)]],
            valv.at[pl.ds(f * _RPW, _RPW)],
            gsem,
        ))
    for cp in scopies:
        cp.wait()
    for cp in hcopies:
        cp.wait()

    def reduce(bb, _):
        base = bb * _L
        acc = valv[pl.ds(base, _L)]
        for f in range(1, _F):
            acc = acc + valv[pl.ds(f * _RPW + base, _L)]
        outv[pl.ds(base, _L)] = acc
        return 0

    lax.fori_loop(0, _BBLK, reduce, 0)

    pltpu.sync_copy(outv, out_hbm.at[pl.ds(base_row, _RPW)])


@jax.jit
def _fmlinear(x_t, table_flat):
    mesh = plsc.VectorSubcoreMesh(
        core_axis_name="c", subcore_axis_name="s",
        num_cores=_NC, num_subcores=_NS,
    )
    return pl.kernel(
        _tec_body,
        out_type=jax.ShapeDtypeStruct((_B,), jnp.float32),
        mesh=mesh,
        scratch_types=[
            pltpu.VMEM((_EPW,), jnp.int32),
            pltpu.VMEM((_EPW,), jnp.float32),
            pltpu.VMEM((_RPW,), jnp.float32),
            pltpu.VMEM_SHARED((_SPW,), jnp.float32),
            pltpu.SemaphoreType.DMA,
            pltpu.SemaphoreType.DMA,
            pltpu.SemaphoreType.DMA,
            pltpu.SemaphoreType.DMA,
        ],
        compiler_params=pltpu.CompilerParams(
            needs_layout_passes=False, disable_bounds_checks=True,
        ),
    )(x_t, table_flat)


def kernel(x, table):
    table_flat = jnp.pad(table, ((0, _TPAD - table.shape[0]), (0, 0))).reshape(-1)
    out = _fmlinear(x.T, table_flat)
    return out.reshape(_B, 1)


# speedup vs baseline: 3.5018x; 1.0108x over previous
import jax
import jax.numpy as jnp
from jax import lax
from jax.experimental import pallas as pl
from jax.experimental.pallas import tpu as pltpu
from jax.experimental.pallas import tpu_sc as plsc

_B = 16384
_F = 26
_FIELD_DIM = 40000
_NC = 2
_NS = 16
_NW = _NC * _NS
_RPW = _B // _NW
_EPW = _RPW * _F
_L = 16
_BBLK = _RPW // _L
_FSP = 17
_CHW = 43008
_NST = 16
_SPW = _NST * _CHW
_TPAD = 1040384


def _tec_body(x_hbm, table_hbm, out_hbm, xv, valv, outv, table_sp, sem, gsem, hsem, tsem):
    sid = lax.axis_index("s")
    wid = sid * _NC + lax.axis_index("c")
    base_row = wid * _RPW

    for f in range(_NST):
        @pl.when(sid == f)
        def _stage(f=f):
            pltpu.async_copy(
                table_hbm.at[pl.ds(f * _CHW, _CHW)],
                table_sp.at[pl.ds(f * _CHW, _CHW)],
                tsem,
            )

    xcopies = [
        pltpu.async_copy(
            x_hbm.at[f, pl.ds(base_row, _RPW)],
            xv.at[pl.ds(f * _RPW, _RPW)],
            sem,
        )
        for f in range(_F)
    ]

    for cp in xcopies:
        cp.wait()
    hcopies = []
    for f in range(_FSP, _F):
        hcopies.append(pltpu.async_copy(
            table_hbm.at[pl.ds(f * _FIELD_DIM, _FIELD_DIM)].at[
                xv.at[pl.ds(f * _RPW, _RPW)]],
            valv.at[pl.ds(f * _RPW, _RPW)],
            hsem,
        ))

    @pl.when(sid < _NST)
    def _drain_stage():
        pltpu.make_async_copy(
            table_hbm.at[pl.ds(0, _CHW)],
            table_sp.at[pl.ds(0, _CHW)],
            tsem,
        ).wait()

    plsc.subcore_barrier()
    scopies = []
    for f in range(_FSP):
        scopies.append(pltpu.async_copy(
            table_sp.at[pl.ds(f * _FIELD_DIM, _FIELD_DIM)].at[
                xv.at[pl.ds(f * _RPW, _RPW)]],
            valv.at[pl.ds(f * _RPW, _RPW)],
            gsem,
        ))
    for cp in scopies:
        cp.wait()
    for cp in hcopies:
        cp.wait()

    def reduce(bb, _):
        base = bb * _L
        acc = valv[pl.ds(base, _L)]
        for f in range(1, _F):
            acc = acc + valv[pl.ds(f * _RPW + base, _L)]
        outv[pl.ds(base, _L)] = acc
        return 0

    lax.fori_loop(0, _BBLK, reduce, 0)

    pltpu.sync_copy(outv, out_hbm.at[pl.ds(base_row, _RPW)])


@jax.jit
def _fmlinear(x_t, table_flat):
    mesh = plsc.VectorSubcoreMesh(
        core_axis_name="c", subcore_axis_name="s",
        num_cores=_NC, num_subcores=_NS,
    )
    return pl.kernel(
        _tec_body,
        out_type=jax.ShapeDtypeStruct((_B,), jnp.float32),
        mesh=mesh,
        scratch_types=[
            pltpu.VMEM((_EPW,), jnp.int32),
            pltpu.VMEM((_EPW,), jnp.float32),
            pltpu.VMEM((_RPW,), jnp.float32),
            pltpu.VMEM_SHARED((_SPW,), jnp.float32),
            pltpu.SemaphoreType.DMA,
            pltpu.SemaphoreType.DMA,
            pltpu.SemaphoreType.DMA,
            pltpu.SemaphoreType.DMA,
        ],
        compiler_params=pltpu.CompilerParams(
            needs_layout_passes=False, disable_bounds_checks=True,
        ),
    )(x_t, table_flat)


def kernel(x, table):
    table_flat = jnp.pad(table, ((0, _TPAD - table.shape[0]), (0, 0))).reshape(-1)
    out = _fmlinear(x.T, table_flat)
    return out.reshape(_B, 1)
